# Initial kernel scaffold; baseline (speedup 1.0000x reference)
import jax
import jax.numpy as jnp
from jax.experimental import pallas as pl


def kernel(x_embed, cls_features, prompt, prompt_key):
    raise NotImplementedError("write your pallas kernel here")


# speedup vs baseline: 1.1426x; 1.1426x over previous
import functools

import jax
import jax.numpy as jnp
from jax import lax
from jax.experimental import pallas as pl
from jax.experimental.pallas import tpu as pltpu
from jax.experimental.pallas import tpu_sc as plsc

_POOL = 1024
_LEN = 8
_DIM = 768
_K = 4
_B = 4
_HALF = _LEN * _DIM // 2


def _gumbel_rows():
    gkey = jax.random.key(42)
    outs = []
    for _ in range(_K):
        gkey, sub = jax.random.split(gkey)
        u = jax.random.uniform(sub, (_B, _POOL), minval=1e-20, maxval=1.0)
        g = -jnp.log(-jnp.log(u) + 1e-20)
        outs.append(jnp.pad(g, ((0, 8 - _B), (0, 0))))
    return outs


def _select_body(pk_ref, q_ref, g0, g1, g2, g3, out_ref):
    pk = pk_ref[...]
    pk = pk * lax.rsqrt(jnp.maximum(jnp.sum(pk * pk, axis=1, keepdims=True), 1e-12))
    qv = q_ref[...]
    qv = qv * lax.rsqrt(jnp.maximum(jnp.sum(qv * qv, axis=1, keepdims=True), 1e-12))
    sim = lax.dot_general(qv, pk, (((1,), (1,)), ((), ())),
                          preferred_element_type=jnp.float32)
    iota = lax.broadcasted_iota(jnp.int32, (8, _POOL), 1)
    cur = sim
    cols = []
    for g_ref in (g0, g1, g2, g3):
        logits = cur + g_ref[...]
        m = jnp.max(logits, axis=1, keepdims=True)
        idx = jnp.min(jnp.where(logits == m, iota, jnp.int32(2 ** 30)),
                      axis=1, keepdims=True)
        cols.append(idx)
        cur = jnp.where(iota == idx, cur - 1000.0, cur)
    out = jnp.concatenate(cols, axis=0)
    out_ref[...] = jnp.broadcast_to(out, (32, 128))


def _gather(idx32, table):
    mesh = plsc.VectorSubcoreMesh(core_axis_name="c", subcore_axis_name="s")

    @functools.partial(
        pl.kernel,
        mesh=mesh,
        out_type=jax.ShapeDtypeStruct((2 * _K * _B, _HALF), jnp.float32),
        scratch_types=[
            pltpu.VMEM((16,), jnp.int32),
            pltpu.VMEM((16,), jnp.int32),
            pltpu.VMEM((1, _HALF), jnp.float32),
            pltpu.SemaphoreType.DMA,
        ],
    )
    def k(idx_hbm, tab_hbm, out_hbm, idx_v, eidx_v, row_v, sem):
        w = lax.axis_index("s") * 2 + lax.axis_index("c")
        r = w >> 3
        b = (w >> 1) & 3
        h = w & 1
        pltpu.sync_copy(idx_hbm.at[r * 8 + b, pl.ds(0, 16)], idx_v)
        eidx_v[...] = idx_v[...] * 2 + h
        pltpu.async_copy(tab_hbm.at[eidx_v.at[pl.ds(0, 1)]], row_v, sem).wait()
        pltpu.sync_copy(row_v, out_hbm.at[pl.ds(b * 8 + r * 2 + h, 1), :])

    return k(idx32, table)


def kernel(x_embed, cls_features, prompt, prompt_key):
    del x_embed
    g0, g1, g2, g3 = _gumbel_rows()
    cls8 = jnp.pad(cls_features, ((0, 8 - _B), (0, 0)))
    idx32 = pl.pallas_call(
        _select_body,
        out_shape=jax.ShapeDtypeStruct((32, 128), jnp.int32),
    )(prompt_key, cls8, g0, g1, g2, g3)
    table = prompt.reshape(2 * _POOL, _HALF)
    rows = _gather(idx32, table)
    return rows.reshape(_B, _K * _LEN, _DIM)


# speedup vs baseline: 1.8902x; 1.6542x over previous
import functools

import jax
import jax.numpy as jnp
from jax import lax
from jax.experimental import pallas as pl
from jax.experimental.pallas import tpu as pltpu
from jax.experimental.pallas import tpu_sc as plsc

_POOL = 1024
_LEN = 8
_DIM = 768
_K = 4
_B = 4
_HALF = _LEN * _DIM // 2

_NOISE_CACHE = []


def _gumbel_noise():
    if not _NOISE_CACHE:
        with jax.ensure_compile_time_eval():
            gkey = jax.random.key(42)
            outs = []
            for _ in range(_K):
                gkey, sub = jax.random.split(gkey)
                u = jax.random.uniform(sub, (_B, _POOL), minval=1e-20, maxval=1.0)
                g = -jnp.log(-jnp.log(u) + 1e-20)
                outs.append(jnp.pad(g, ((0, 8 - _B), (0, 0))))
            _NOISE_CACHE.append(jnp.concatenate(outs, axis=0))
    return _NOISE_CACHE[0]


def _select_body(pk_ref, q_ref, g_ref, out_ref):
    pk = pk_ref[...]
    pk = pk * lax.rsqrt(jnp.maximum(jnp.sum(pk * pk, axis=1, keepdims=True), 1e-12))
    qv = q_ref[...]
    qv = qv * lax.rsqrt(jnp.maximum(jnp.sum(qv * qv, axis=1, keepdims=True), 1e-12))
    sim = lax.dot_general(qv, pk, (((1,), (1,)), ((), ())),
                          preferred_element_type=jnp.float32)
    iota = lax.broadcasted_iota(jnp.int32, (_B, _POOL), 1)
    cur = sim
    cols = []
    for r in range(_K):
        logits = cur + g_ref[pl.ds(8 * r, _B), :]
        m = jnp.max(logits, axis=1, keepdims=True)
        idx = jnp.min(jnp.where(logits == m, iota, jnp.int32(2 ** 30)),
                      axis=1, keepdims=True)
        cols.append(idx)
        cur = jnp.where(iota == idx, cur - 1000.0, cur)
    out = jnp.concatenate(cols, axis=0)
    out_ref[...] = jnp.broadcast_to(out, (_K * _B, 128))


def _gather(idx16, table):
    mesh = plsc.VectorSubcoreMesh(core_axis_name="c", subcore_axis_name="s")

    @functools.partial(
        pl.kernel,
        mesh=mesh,
        out_type=jax.ShapeDtypeStruct((2 * _K * _B, _HALF), jnp.float32),
        scratch_types=[
            pltpu.VMEM((16,), jnp.int32),
            pltpu.VMEM((16,), jnp.int32),
            pltpu.VMEM((1, _HALF), jnp.float32),
            pltpu.SemaphoreType.DMA,
        ],
    )
    def k(idx_hbm, tab_hbm, out_hbm, idx_v, eidx_v, row_v, sem):
        w = lax.axis_index("s") * 2 + lax.axis_index("c")
        p = w >> 1
        h = w & 1
        r = p >> 2
        b = p & 3
        pltpu.sync_copy(idx_hbm.at[p, pl.ds(0, 16)], idx_v)
        eidx_v[...] = idx_v[...] * 2 + h
        pltpu.async_copy(tab_hbm.at[eidx_v.at[pl.ds(0, 1)]], row_v, sem).wait()
        pltpu.sync_copy(row_v, out_hbm.at[pl.ds(b * 8 + r * 2 + h, 1), :])

    return k(idx16, table)


def kernel(x_embed, cls_features, prompt, prompt_key):
    del x_embed
    idx16 = pl.pallas_call(
        _select_body,
        out_shape=jax.ShapeDtypeStruct((_K * _B, 128), jnp.int32),
    )(prompt_key, cls_features, _gumbel_noise())
    table = prompt.reshape(2 * _POOL, _HALF)
    rows = _gather(idx16, table)
    return rows.reshape(_B, _K * _LEN, _DIM)
